# baseline (speedup 1.0000x reference)
import functools

import jax
import jax.numpy as jnp
from jax import lax
from jax.experimental import pallas as pl
from jax.experimental.pallas import tpu as pltpu
from jax.experimental.pallas import tpu_sc as plsc

IMAGE_SIZE = 128
NEAR = 0.1
FAR = 100.0
INV_FAR = 1.0 / FAR
INV_NEAR = 1.0 / NEAR

NC, NS = 2, 16
NW = NC * NS
F = 1024
FPW = F // NS
NPIX = IMAGE_SIZE * IMAGE_SIZE
SLICE = NPIX // NS


def _sc_raster(faces_hbm, out_hbm, fbuf, dbuf, xpbuf, mbuf, obuf, shared):
    c = lax.axis_index("c")
    s = lax.axis_index("s")
    wid = c * NS + s

    pltpu.sync_copy(faces_hbm.at[pl.ds(wid * (FPW * 9), FPW * 9 + 8)], fbuf)

    lane = lax.iota(jnp.int32, 16)
    for cc in range(IMAGE_SIZE // 16):
        xi = (lane + cc * 16).astype(jnp.float32)
        xpbuf[pl.ds(cc * 16, 16)] = (2.0 * xi + 1.0 - IMAGE_SIZE) * (1.0 / IMAGE_SIZE)

    bg = jnp.full((16,), INV_FAR, dtype=jnp.float32)

    def clear_body(i, _):
        for u in range(8):
            dbuf[pl.ds(i * 128 + u * 16, 16)] = bg
        return 0

    lax.fori_loop(0, NPIX // 128, clear_body, 0)

    def face_body(f, _):
        v = fbuf[pl.ds(f * 9, 16)]
        x0 = v[0]; y0 = v[1]; z0 = v[2]
        x1 = v[3]; y1 = v[4]; z1 = v[5]
        x2 = v[6]; y2 = v[7]; z2 = v[8]

        half = IMAGE_SIZE // 2
        xmn = jnp.minimum(jnp.minimum(x0, x1), x2)
        xmx = jnp.maximum(jnp.maximum(x0, x1), x2)
        ymn = jnp.minimum(jnp.minimum(y0, y1), y2)
        ymx = jnp.maximum(jnp.maximum(y0, y1), y2)
        i_lo = jnp.clip((xmn + 1.0) * half - 0.5, 0.0, 127.0).astype(jnp.int32)
        i_hi = jnp.clip((xmx + 1.0) * half - 0.5, -2.0, 127.0).astype(jnp.int32) + 1
        i_hi = jnp.minimum(i_hi, 127)
        j_lo = jnp.clip((ymn + 1.0) * half - 0.5, 0.0, 127.0).astype(jnp.int32)
        j_hi = jnp.clip((ymx + 1.0) * half - 0.5, -2.0, 127.0).astype(jnp.int32) + 1
        j_hi = jnp.minimum(j_hi, 127)
        nchunks = ((i_hi - i_lo) >> 4) + 1

        x0v = jnp.full((16,), x0, jnp.float32)
        y0v = jnp.full((16,), y0, jnp.float32)
        x1v = jnp.full((16,), x1, jnp.float32)
        y1v = jnp.full((16,), y1, jnp.float32)
        x2v = jnp.full((16,), x2, jnp.float32)
        y2v = jnp.full((16,), y2, jnp.float32)
        r0v = 1.0 / jnp.full((16,), z0, jnp.float32)
        r1v = 1.0 / jnp.full((16,), z1, jnp.float32)
        r2v = 1.0 / jnp.full((16,), z2, jnp.float32)

        def eval_row(dy0, dy1, dy2, dx0, dx1, dx2):
            w0 = dx1 * dy2 - dy1 * dx2
            w1 = dx2 * dy0 - dy2 * dx0
            w2 = dx0 * dy1 - dy0 * dx1
            wsum = w0 + w1 + w2
            inside = ((w0 * w1) > 0) & ((w0 * w2) > 0)
            wsum_safe = jnp.where(jnp.abs(wsum) < 1e-12, 1.0, wsum)
            inv_z = (w0 * r0v + w1 * r1v + w2 * r2v) / wsum_safe
            valid = inside & (inv_z < INV_NEAR)
            return jnp.where(valid, inv_z, INV_FAR)

        def rowtri_body(jj, _):
            j = j_lo + 3 * jj
            j2 = jnp.minimum(j + 1, 127)
            j3 = jnp.minimum(j + 2, 127)
            ypa = (2.0 * j.astype(jnp.float32) + 1.0 - IMAGE_SIZE) * (1.0 / IMAGE_SIZE)
            ypb = (2.0 * j2.astype(jnp.float32) + 1.0 - IMAGE_SIZE) * (1.0 / IMAGE_SIZE)
            ypc = (2.0 * j3.astype(jnp.float32) + 1.0 - IMAGE_SIZE) * (1.0 / IMAGE_SIZE)
            dy0a = y0v - ypa; dy1a = y1v - ypa; dy2a = y2v - ypa
            dy0b = y0v - ypb; dy1b = y1v - ypb; dy2b = y2v - ypb
            dy0c = y0v - ypc; dy1c = y1v - ypc; dy2c = y2v - ypc
            rowa = j * IMAGE_SIZE
            rowb = j2 * IMAGE_SIZE
            rowc = j3 * IMAGE_SIZE

            def chunk_body(cc, _):
                xb = jnp.minimum(i_lo + cc * 16, IMAGE_SIZE - 16)
                xp = xpbuf[pl.ds(xb, 16)]
                dx0 = x0v - xp
                dx1 = x1v - xp
                dx2 = x2v - xp
                qa = eval_row(dy0a, dy1a, dy2a, dx0, dx1, dx2)
                qb = eval_row(dy0b, dy1b, dy2b, dx0, dx1, dx2)
                qc = eval_row(dy0c, dy1c, dy2c, dx0, dx1, dx2)
                cura = dbuf[pl.ds(rowa + xb, 16)]
                dbuf[pl.ds(rowa + xb, 16)] = jnp.maximum(cura, qa)
                curb = dbuf[pl.ds(rowb + xb, 16)]
                dbuf[pl.ds(rowb + xb, 16)] = jnp.maximum(curb, qb)
                curc = dbuf[pl.ds(rowc + xb, 16)]
                dbuf[pl.ds(rowc + xb, 16)] = jnp.maximum(curc, qc)
                return 0

            lax.fori_loop(0, nchunks, chunk_body, 0)
            return 0

        ntri = ((j_hi - j_lo + 3) * 21846) >> 16
        lax.fori_loop(0, ntri, rowtri_body, 0)
        return 0

    lax.fori_loop(0, FPW, face_body, 0)

    pltpu.sync_copy(dbuf, shared.at[s])
    plsc.subcore_barrier()
    pltpu.sync_copy(shared.at[:, pl.ds(s * SLICE, SLICE)], mbuf)

    def merge_body(i, _):
        q = mbuf[0, pl.ds(i * 16, 16)]
        for k in range(1, NS):
            q = jnp.maximum(q, mbuf[k, pl.ds(i * 16, 16)])
        obuf[pl.ds(i * 16, 16)] = 1.0 / q
        return 0

    lax.fori_loop(0, SLICE // 16, merge_body, 0)

    pltpu.sync_copy(obuf, out_hbm.at[c, pl.ds(s * SLICE, SLICE)])


def _sc_rasterize(comp):
    mesh = plsc.VectorSubcoreMesh(
        core_axis_name="c", subcore_axis_name="s", num_cores=NC, num_subcores=NS
    )
    run = functools.partial(
        pl.kernel,
        out_type=jax.ShapeDtypeStruct((2, NPIX), jnp.float32),
        mesh=mesh,
        scratch_types=[
            pltpu.VMEM((FPW * 9 + 8,), jnp.float32),
            pltpu.VMEM((NPIX,), jnp.float32),
            pltpu.VMEM((IMAGE_SIZE,), jnp.float32),
            pltpu.VMEM((NS, SLICE), jnp.float32),
            pltpu.VMEM((SLICE,), jnp.float32),
            pltpu.VMEM_SHARED((NS, NPIX), jnp.float32),
        ],
    )(_sc_raster)
    return run(comp)


def kernel(input):
    faces = input
    comp = jnp.pad(faces.reshape(NW * FPW * 9), (0, 8))
    return _sc_rasterize(comp).reshape(2, IMAGE_SIZE, IMAGE_SIZE)


# speedup vs baseline: 1.5279x; 1.5279x over previous
import functools

import jax
import jax.numpy as jnp
from jax import lax
from jax.experimental import pallas as pl
from jax.experimental.pallas import tpu as pltpu
from jax.experimental.pallas import tpu_sc as plsc

IMAGE_SIZE = 128
NEAR = 0.1
FAR = 100.0
INV_FAR = 1.0 / FAR
INV_NEAR = 1.0 / NEAR

NC, NS = 2, 16
NW = NC * NS
F = 1024
FPW = F // NS
NPIX = IMAGE_SIZE * IMAGE_SIZE
SLICE = NPIX // NS


def _sc_raster(faces_hbm, out_hbm, fbuf, dbuf, xpbuf, mbuf, obuf, shared):
    c = lax.axis_index("c")
    s = lax.axis_index("s")
    wid = c * NS + s

    pltpu.sync_copy(faces_hbm.at[pl.ds(wid * (FPW * 9), FPW * 9 + 8)], fbuf)

    lane = lax.iota(jnp.int32, 16)
    for cc in range(IMAGE_SIZE // 16):
        xi = (lane + cc * 16).astype(jnp.float32)
        xpbuf[pl.ds(cc * 16, 16)] = (2.0 * xi + 1.0 - IMAGE_SIZE) * (1.0 / IMAGE_SIZE)

    bg = jnp.full((16,), INV_FAR, dtype=jnp.float32)

    def clear_body(i, _):
        for u in range(8):
            dbuf[pl.ds(i * 128 + u * 16, 16)] = bg
        return 0

    lax.fori_loop(0, NPIX // 128, clear_body, 0)

    def face_body(f, _):
        v = fbuf[pl.ds(f * 9, 16)]
        x0 = v[0]; y0 = v[1]; z0 = v[2]
        x1 = v[3]; y1 = v[4]; z1 = v[5]
        x2 = v[6]; y2 = v[7]; z2 = v[8]

        half = IMAGE_SIZE // 2
        xmn = jnp.minimum(jnp.minimum(x0, x1), x2)
        xmx = jnp.maximum(jnp.maximum(x0, x1), x2)
        ymn = jnp.minimum(jnp.minimum(y0, y1), y2)
        ymx = jnp.maximum(jnp.maximum(y0, y1), y2)
        i_lo = jnp.clip((xmn + 1.0) * half - 0.5, 0.0, 127.0).astype(jnp.int32)
        i_hi = jnp.clip((xmx + 1.0) * half - 0.5, -2.0, 127.0).astype(jnp.int32) + 1
        i_hi = jnp.minimum(i_hi, 127)
        j_lo = jnp.clip((ymn + 1.0) * half - 0.5, 0.0, 127.0).astype(jnp.int32)
        j_hi = jnp.clip((ymx + 1.0) * half - 0.5, -2.0, 127.0).astype(jnp.int32) + 1
        j_hi = jnp.minimum(j_hi, 127)
        nchunks = ((i_hi - i_lo) >> 4) + 1

        x0v = jnp.full((16,), x0, jnp.float32)
        y0v = jnp.full((16,), y0, jnp.float32)
        x1v = jnp.full((16,), x1, jnp.float32)
        y1v = jnp.full((16,), y1, jnp.float32)
        x2v = jnp.full((16,), x2, jnp.float32)
        y2v = jnp.full((16,), y2, jnp.float32)
        r0v = 1.0 / jnp.full((16,), z0, jnp.float32)
        r1v = 1.0 / jnp.full((16,), z1, jnp.float32)
        r2v = 1.0 / jnp.full((16,), z2, jnp.float32)

        def eval_row(dy0, dy1, dy2, dx0, dx1, dx2):
            w0 = dx1 * dy2 - dy1 * dx2
            w1 = dx2 * dy0 - dy2 * dx0
            w2 = dx0 * dy1 - dy0 * dx1
            wsum = w0 + w1 + w2
            inside = ((w0 * w1) > 0) & ((w0 * w2) > 0)
            wsum_safe = jnp.where(jnp.abs(wsum) < 1e-12, 1.0, wsum)
            inv_z = (w0 * r0v + w1 * r1v + w2 * r2v) / wsum_safe
            valid = inside & (inv_z < INV_NEAR)
            return jnp.where(valid, inv_z, INV_FAR)

        def rowpair_body(jj, _):
            j = j_lo + 2 * jj
            j2 = jnp.minimum(j + 1, 127)
            ypa = (2.0 * j.astype(jnp.float32) + 1.0 - IMAGE_SIZE) * (1.0 / IMAGE_SIZE)
            ypb = (2.0 * j2.astype(jnp.float32) + 1.0 - IMAGE_SIZE) * (1.0 / IMAGE_SIZE)
            dy0a = y0v - ypa; dy1a = y1v - ypa; dy2a = y2v - ypa
            dy0b = y0v - ypb; dy1b = y1v - ypb; dy2b = y2v - ypb
            rowa = j * IMAGE_SIZE
            rowb = j2 * IMAGE_SIZE

            def chunk_body(cc, _):
                xb = jnp.minimum(i_lo + cc * 16, IMAGE_SIZE - 16)
                xp = xpbuf[pl.ds(xb, 16)]
                dx0 = x0v - xp
                dx1 = x1v - xp
                dx2 = x2v - xp
                qa = eval_row(dy0a, dy1a, dy2a, dx0, dx1, dx2)
                qb = eval_row(dy0b, dy1b, dy2b, dx0, dx1, dx2)
                cura = dbuf[pl.ds(rowa + xb, 16)]
                dbuf[pl.ds(rowa + xb, 16)] = jnp.maximum(cura, qa)
                curb = dbuf[pl.ds(rowb + xb, 16)]
                dbuf[pl.ds(rowb + xb, 16)] = jnp.maximum(curb, qb)
                return 0

            lax.fori_loop(0, nchunks, chunk_body, 0)
            return 0

        npairs = (j_hi - j_lo + 2) >> 1
        lax.fori_loop(0, npairs, rowpair_body, 0)
        return 0

    lax.fori_loop(0, FPW, face_body, 0)

    pltpu.sync_copy(dbuf, shared.at[s])
    plsc.subcore_barrier()
    pltpu.sync_copy(shared.at[:, pl.ds(s * SLICE, SLICE)], mbuf)

    def merge_body(i, _):
        q = mbuf[0, pl.ds(i * 16, 16)]
        for k in range(1, NS):
            q = jnp.maximum(q, mbuf[k, pl.ds(i * 16, 16)])
        obuf[pl.ds(i * 16, 16)] = 1.0 / q
        return 0

    lax.fori_loop(0, SLICE // 16, merge_body, 0)

    pltpu.sync_copy(obuf, out_hbm.at[c, pl.ds(s * SLICE, SLICE)])


def _sc_rasterize(comp):
    mesh = plsc.VectorSubcoreMesh(
        core_axis_name="c", subcore_axis_name="s", num_cores=NC, num_subcores=NS
    )
    run = functools.partial(
        pl.kernel,
        out_type=jax.ShapeDtypeStruct((2, NPIX), jnp.float32),
        mesh=mesh,
        scratch_types=[
            pltpu.VMEM((FPW * 9 + 8,), jnp.float32),
            pltpu.VMEM((NPIX,), jnp.float32),
            pltpu.VMEM((IMAGE_SIZE,), jnp.float32),
            pltpu.VMEM((NS, SLICE), jnp.float32),
            pltpu.VMEM((SLICE,), jnp.float32),
            pltpu.VMEM_SHARED((NS, NPIX), jnp.float32),
        ],
    )(_sc_raster)
    return run(comp)


def kernel(input):
    faces = input
    comp = jnp.pad(faces.reshape(NW * FPW * 9), (0, 8))
    return _sc_rasterize(comp).reshape(2, IMAGE_SIZE, IMAGE_SIZE)
